# baseline (speedup 1.0000x reference)
import functools

import jax
import jax.numpy as jnp
from jax import lax
from jax.experimental import pallas as pl
from jax.experimental.pallas import tpu as pltpu
from jax.experimental.pallas import tpu_sc as plsc

N_FACTORS = 64
BATCH = 16384
NC, NS, L = 2, 16, 16
NW = NC * NS
B_PER_W = BATCH // NW
CHUNK = 128
N_CHUNKS = B_PER_W // CHUNK
GROUPS = B_PER_W // L


def _sc_body(u_hbm, m_hbm, data_hbm, out_hbm,
             data_v, uidx_v, midx_v, u_rows, m_rows, out_v, sem):
    wid = lax.axis_index("s") * NC + lax.axis_index("c")
    base = wid * B_PER_W

    pltpu.sync_copy(data_hbm.at[wid], data_v)

    lane = lax.iota(jnp.int32, L)
    two_lane = lane * 2

    @plsc.parallel_loop(0, B_PER_W, L)
    def _deint(i):
        b2 = 2 * i + two_lane
        uidx_v[pl.ds(i, L)] = plsc.load_gather(data_v, [b2])
        midx_v[pl.ds(i, L)] = plsc.load_gather(data_v, [b2 + 1])

    copies = []
    for c in range(N_CHUNKS):
        copies.append(pltpu.make_async_copy(
            u_hbm.at[uidx_v.at[pl.ds(c * CHUNK, CHUNK)]],
            u_rows.at[pl.ds(c * CHUNK, CHUNK)], sem))
        copies.append(pltpu.make_async_copy(
            m_hbm.at[midx_v.at[pl.ds(c * CHUNK, CHUNK)]],
            m_rows.at[pl.ds(c * CHUNK, CHUNK)], sem))
    for cp in copies:
        cp.start()
    for cp in copies:
        cp.wait()

    @plsc.parallel_loop(0, GROUPS, 1)
    def _group(g):
        row = g * L + lane
        accs = [jnp.zeros((L,), jnp.float32) for _ in range(4)]
        for d in range(N_FACTORS):
            col = jnp.full((L,), d, jnp.int32)
            uu = plsc.load_gather(u_rows, [row, col])
            mm = plsc.load_gather(m_rows, [row, col])
            accs[d & 3] = accs[d & 3] + uu * mm
        out_v[pl.ds(g * L, L)] = (accs[0] + accs[1]) + (accs[2] + accs[3])

    pltpu.sync_copy(out_v, out_hbm.at[pl.ds(base, B_PER_W)])


@jax.jit
def kernel(data, user_factors, movie_factors):
    data_r = data.reshape(NW, B_PER_W * 2)
    mesh = plsc.VectorSubcoreMesh(core_axis_name="c", subcore_axis_name="s")
    f = pl.kernel(
        _sc_body,
        out_type=jax.ShapeDtypeStruct((BATCH,), jnp.float32),
        mesh=mesh,
        scratch_types=[
            pltpu.VMEM((B_PER_W * 2,), jnp.int32),
            pltpu.VMEM((B_PER_W,), jnp.int32),
            pltpu.VMEM((B_PER_W,), jnp.int32),
            pltpu.VMEM((B_PER_W, N_FACTORS), jnp.float32),
            pltpu.VMEM((B_PER_W, N_FACTORS), jnp.float32),
            pltpu.VMEM((B_PER_W,), jnp.float32),
            pltpu.SemaphoreType.DMA,
        ],
        compiler_params=pltpu.CompilerParams(
            needs_layout_passes=False, use_tc_tiling_on_sc=False),
    )
    return f(user_factors, movie_factors, data_r)
